# baseline (speedup 1.0000x reference)
import functools

import jax
import jax.numpy as jnp
from jax import lax
from jax.experimental import pallas as pl
from jax.experimental.pallas import tpu as pltpu
from jax.experimental.pallas import tpu_sc as plsc

TOP_K = 5


def _sc_gather(table, ids_flat):
    V, D = table.shape
    BL = ids_flat.shape[0]
    info = plsc.get_sparse_core_info()
    nw = info.num_cores * info.num_subcores
    b_per_w = BL // nw
    nc = info.num_cores
    mesh = plsc.VectorSubcoreMesh(core_axis_name="c", subcore_axis_name="s")

    @functools.partial(
        pl.kernel,
        out_type=jax.ShapeDtypeStruct((BL, D), jnp.float32),
        mesh=mesh,
        scratch_types=[
            pltpu.VMEM((b_per_w,), jnp.int32),
            pltpu.VMEM((b_per_w, D), jnp.float32),
            pltpu.SemaphoreType.DMA,
        ],
    )
    def gather_k(table_hbm, idx_hbm, out_hbm, idx_v, rows_v, sem):
        wid = lax.axis_index("s") * nc + lax.axis_index("c")
        base = wid * b_per_w
        pltpu.sync_copy(idx_hbm.at[pl.ds(base, b_per_w)], idx_v)
        pltpu.async_copy(table_hbm.at[idx_v], rows_v, sem).wait()
        pltpu.sync_copy(rows_v, out_hbm.at[pl.ds(base, b_per_w)])

    return gather_k(table, ids_flat)


def _trunc_hi(x):
    xi = jax.lax.bitcast_convert_type(x, jnp.uint32)
    return jax.lax.bitcast_convert_type(xi & jnp.uint32(0xFFFF0000), jnp.float32)


def _split_dot(s_row, a):
    s_hi = _trunc_hi(s_row)
    s2 = jnp.concatenate([s_hi, s_row - s_hi], axis=0)
    a_hi = _trunc_hi(a)
    a_lo = a - a_hi
    dims = (((1,), (1,)), ((), ()))

    def d(x, y):
        return jax.lax.dot_general(x, y, dimension_numbers=dims,
                                   preferred_element_type=jnp.float32)

    r_hi = d(s2, a_hi)
    r_lo = d(s2, a_lo)
    return ((r_lo[1:2] + r_lo[0:1]) + r_hi[1:2]) + r_hi[0:1]


def _score_body(tok_ref, w_ref, mask_ref, g_ref, alpha_ref, a_ref,
                logits_ref, action_ref, *, B, N, NB, D, top_k):
    b = pl.program_id(0)
    j = pl.program_id(1)
    nb = pl.num_programs(0)
    nj = pl.num_programs(1)

    w_row = w_ref[0]
    tok_b = tok_ref[0]
    s_row = jax.lax.dot_general(
        w_row, tok_b, dimension_numbers=(((1,), (0,)), ((), ())),
        precision=jax.lax.Precision.HIGHEST,
        preferred_element_type=jnp.float32)

    a = a_ref[0]
    logits_ref[pl.ds(b, 1), pl.ds(j * NB, NB)] = _split_dot(s_row, a)

    @pl.when((b == nb - 1) & (j == nj - 1))
    def _():
        raw = logits_ref[...]
        gmin = jnp.min(raw)
        avail = mask_ref[...]
        lm = jnp.where(avail, raw, gmin - 1.0)
        logits_ref[...] = lm

        iota_n = jax.lax.broadcasted_iota(jnp.int32, (B, N), 1)
        work = lm
        sel = jnp.zeros((B, N), dtype=jnp.bool_)
        for _ in range(top_k):
            m = jnp.max(work, axis=-1, keepdims=True)
            idx = jnp.min(jnp.where(work == m, iota_n, N), axis=-1, keepdims=True)
            pick = iota_n == idx
            sel = sel | pick
            work = jnp.where(pick, -jnp.inf, work)
        sel = sel & avail

        alpha = alpha_ref[0, 0]
        row_max = jnp.max(lm, axis=-1, keepdims=True)
        e = jnp.where(sel, jnp.exp((lm - row_max) / alpha), 0.0)
        denom = jnp.sum(e, axis=-1, keepdims=True)
        p = e / denom
        log_p = jnp.where(p > 0, jnp.log(jnp.clip(p, 1e-30)), -1e30)
        score = log_p + g_ref[...]
        smax = jnp.max(score, axis=-1, keepdims=True)
        act = jnp.min(jnp.where(score == smax, iota_n, N), axis=-1)
        action_ref[...] = act.reshape(1, B)


def kernel(input_ids, attention_mask, available_mask, a_embeds, embed_table, alpha):
    B, L = input_ids.shape
    _, N, D = a_embeds.shape

    g = jax.random.gumbel(jax.random.key(42), (B, N), jnp.float32)

    w = attention_mask / jnp.maximum(
        attention_mask.sum(axis=-1, keepdims=True), 1e-6)
    w = w.astype(jnp.float32)

    ids_flat = input_ids.reshape(B * L).astype(jnp.int32)
    tok = _sc_gather(embed_table, ids_flat).reshape(B, L, D)

    NJ = 2
    NB = N // NJ
    logits, action = pl.pallas_call(
        functools.partial(_score_body, B=B, N=N, NB=NB, D=D,
                          top_k=min(N, TOP_K)),
        grid=(B, NJ),
        in_specs=[
            pl.BlockSpec((1, L, D), lambda b, j: (b, 0, 0)),
            pl.BlockSpec((1, 1, L), lambda b, j: (b, 0, 0)),
            pl.BlockSpec((B, N), lambda b, j: (0, 0)),
            pl.BlockSpec((B, N), lambda b, j: (0, 0)),
            pl.BlockSpec(memory_space=pltpu.SMEM),
            pl.BlockSpec((1, NB, D), lambda b, j: (b, j, 0)),
        ],
        out_specs=[
            pl.BlockSpec((B, N), lambda b, j: (0, 0)),
            pl.BlockSpec((1, B), lambda b, j: (0, 0)),
        ],
        out_shape=[
            jax.ShapeDtypeStruct((B, N), jnp.float32),
            jax.ShapeDtypeStruct((1, B), jnp.int32),
        ],
    )(tok, w.reshape(B, 1, L), available_mask, g, alpha.reshape(1, 1), a_embeds)

    return (action.reshape(B), logits)


# speedup vs baseline: 1.4165x; 1.4165x over previous
import functools

import jax
import jax.numpy as jnp
from jax import lax
from jax.experimental import pallas as pl
from jax.experimental.pallas import tpu as pltpu
from jax.experimental.pallas import tpu_sc as plsc

TOP_K = 5


def _sc_gather(table, ids_flat):
    V, D = table.shape
    BL = ids_flat.shape[0]
    info = plsc.get_sparse_core_info()
    nw = info.num_cores * info.num_subcores
    b_per_w = BL // nw
    nc = info.num_cores
    mesh = plsc.VectorSubcoreMesh(core_axis_name="c", subcore_axis_name="s")

    @functools.partial(
        pl.kernel,
        out_type=jax.ShapeDtypeStruct((BL, D), jnp.float32),
        mesh=mesh,
        scratch_types=[
            pltpu.VMEM((b_per_w,), jnp.int32),
            pltpu.VMEM((b_per_w, D), jnp.float32),
            pltpu.SemaphoreType.DMA,
        ],
    )
    def gather_k(table_hbm, idx_hbm, out_hbm, idx_v, rows_v, sem):
        wid = lax.axis_index("s") * nc + lax.axis_index("c")
        base = wid * b_per_w
        pltpu.sync_copy(idx_hbm.at[pl.ds(base, b_per_w)], idx_v)
        pltpu.async_copy(table_hbm.at[idx_v], rows_v, sem).wait()
        pltpu.sync_copy(rows_v, out_hbm.at[pl.ds(base, b_per_w)])

    return gather_k(table, ids_flat)


def _trunc_hi(x):
    xi = jax.lax.bitcast_convert_type(x, jnp.uint32)
    return jax.lax.bitcast_convert_type(xi & jnp.uint32(0xFFFF0000), jnp.float32)


def _split_dot(s_row, a):
    s_hi = _trunc_hi(s_row)
    s2 = jnp.concatenate([s_hi, s_row - s_hi], axis=0)
    a_hi = _trunc_hi(a)
    a_lo = a - a_hi
    dims = (((1,), (1,)), ((), ()))

    def d(x, y):
        return jax.lax.dot_general(x, y, dimension_numbers=dims,
                                   preferred_element_type=jnp.float32)

    r_hi = d(s2, a_hi)
    r_lo = d(s2, a_lo)
    return ((r_lo[1:2] + r_lo[0:1]) + r_hi[1:2]) + r_hi[0:1]


def _score_body(tok_ref, w_ref, mask_ref, g_ref, alpha_ref, a_ref,
                logits_ref, action_ref, *, B, N, D, GB, top_k):
    b = pl.program_id(0)
    nb = pl.num_programs(0)

    for i in range(GB):
        w_row = w_ref[i]
        tok_b = tok_ref[i]
        s_row = jax.lax.dot_general(
            w_row, tok_b, dimension_numbers=(((1,), (0,)), ((), ())),
            precision=jax.lax.Precision.HIGHEST,
            preferred_element_type=jnp.float32)

        a = a_ref[i]
        logits_ref[pl.ds(b * GB + i, 1), :] = _split_dot(s_row, a)

    @pl.when(b == nb - 1)
    def _():
        raw = logits_ref[...]
        gmin = jnp.min(raw)
        avail = mask_ref[...]
        lm = jnp.where(avail, raw, gmin - 1.0)
        logits_ref[...] = lm

        iota_n = jax.lax.broadcasted_iota(jnp.int32, (B, N), 1)
        work = lm
        sel = jnp.zeros((B, N), dtype=jnp.bool_)
        for _ in range(top_k):
            m = jnp.max(work, axis=-1, keepdims=True)
            idx = jnp.min(jnp.where(work == m, iota_n, N), axis=-1, keepdims=True)
            pick = iota_n == idx
            sel = sel | pick
            work = jnp.where(pick, -jnp.inf, work)
        sel = sel & avail

        alpha = alpha_ref[0, 0]
        row_max = jnp.max(lm, axis=-1, keepdims=True)
        e = jnp.where(sel, jnp.exp((lm - row_max) / alpha), 0.0)
        denom = jnp.sum(e, axis=-1, keepdims=True)
        p = e / denom
        log_p = jnp.where(p > 0, jnp.log(jnp.clip(p, 1e-30)), -1e30)
        score = log_p + g_ref[...]
        smax = jnp.max(score, axis=-1, keepdims=True)
        act = jnp.min(jnp.where(score == smax, iota_n, N), axis=-1)
        action_ref[...] = act.reshape(1, B)


def kernel(input_ids, attention_mask, available_mask, a_embeds, embed_table, alpha):
    B, L = input_ids.shape
    _, N, D = a_embeds.shape

    g = jax.random.gumbel(jax.random.key(42), (B, N), jnp.float32)

    w = attention_mask / jnp.maximum(
        attention_mask.sum(axis=-1, keepdims=True), 1e-6)
    w = w.astype(jnp.float32)

    ids_flat = input_ids.reshape(B * L).astype(jnp.int32)
    tok = _sc_gather(embed_table, ids_flat).reshape(B, L, D)

    GB = 2
    logits, action = pl.pallas_call(
        functools.partial(_score_body, B=B, N=N, D=D, GB=GB,
                          top_k=min(N, TOP_K)),
        grid=(B // GB,),
        in_specs=[
            pl.BlockSpec((GB, L, D), lambda b: (b, 0, 0)),
            pl.BlockSpec((GB, 1, L), lambda b: (b, 0, 0)),
            pl.BlockSpec((B, N), lambda b: (0, 0)),
            pl.BlockSpec((B, N), lambda b: (0, 0)),
            pl.BlockSpec(memory_space=pltpu.SMEM),
            pl.BlockSpec((GB, N, D), lambda b: (b, 0, 0)),
        ],
        out_specs=[
            pl.BlockSpec((B, N), lambda b: (0, 0)),
            pl.BlockSpec((1, B), lambda b: (0, 0)),
        ],
        out_shape=[
            jax.ShapeDtypeStruct((B, N), jnp.float32),
            jax.ShapeDtypeStruct((1, B), jnp.int32),
        ],
    )(tok, w.reshape(B, 1, L), available_mask, g, alpha.reshape(1, 1), a_embeds)

    return (action.reshape(B), logits)


# speedup vs baseline: 1.4875x; 1.0501x over previous
import functools

import jax
import jax.numpy as jnp
from jax import lax
from jax.experimental import pallas as pl
from jax.experimental.pallas import tpu as pltpu
from jax.experimental.pallas import tpu_sc as plsc

TOP_K = 5


def _sc_gather(table, ids_flat):
    V, D = table.shape
    BL = ids_flat.shape[0]
    info = plsc.get_sparse_core_info()
    nw = info.num_cores * info.num_subcores
    b_per_w = BL // nw
    nc = info.num_cores
    mesh = plsc.VectorSubcoreMesh(core_axis_name="c", subcore_axis_name="s")

    @functools.partial(
        pl.kernel,
        out_type=jax.ShapeDtypeStruct((BL, D), jnp.float32),
        mesh=mesh,
        scratch_types=[
            pltpu.VMEM((b_per_w,), jnp.int32),
            pltpu.VMEM((b_per_w, D), jnp.float32),
            pltpu.SemaphoreType.DMA,
        ],
    )
    def gather_k(table_hbm, idx_hbm, out_hbm, idx_v, rows_v, sem):
        wid = lax.axis_index("s") * nc + lax.axis_index("c")
        base = wid * b_per_w
        pltpu.sync_copy(idx_hbm.at[pl.ds(base, b_per_w)], idx_v)
        pltpu.async_copy(table_hbm.at[idx_v], rows_v, sem).wait()
        pltpu.sync_copy(rows_v, out_hbm.at[pl.ds(base, b_per_w)])

    return gather_k(table, ids_flat)


def _trunc_hi(x):
    xi = jax.lax.bitcast_convert_type(x, jnp.uint32)
    return jax.lax.bitcast_convert_type(xi & jnp.uint32(0xFFFF0000), jnp.float32)


def _split_dot(s_row, a):
    s_hi = _trunc_hi(s_row)
    s2 = jnp.concatenate([s_hi, s_row - s_hi], axis=0)
    a_hi = _trunc_hi(a)
    a_lo = a - a_hi
    dims = (((1,), (1,)), ((), ()))

    def d(x, y):
        return jax.lax.dot_general(x, y, dimension_numbers=dims,
                                   preferred_element_type=jnp.float32)

    r_hi = d(s2, a_hi)
    r_lo = d(s2, a_lo)
    return ((r_lo[1:2] + r_lo[0:1]) + r_hi[1:2]) + r_hi[0:1]


def _score_body(tok_ref, w_ref, mask_ref, g_ref, alpha_ref, a_ref,
                logits_ref, action_ref, *, B, N, D, GB, top_k):
    b = pl.program_id(0)
    nb = pl.num_programs(0)

    for i in range(GB):
        w_row = w_ref[i]
        tok_b = tok_ref[i]
        s_row = jax.lax.dot_general(
            w_row, tok_b, dimension_numbers=(((1,), (0,)), ((), ())),
            precision=jax.lax.Precision.HIGHEST,
            preferred_element_type=jnp.float32)

        a = a_ref[i]
        logits_ref[pl.ds(b * GB + i, 1), :] = _split_dot(s_row, a)

    @pl.when(b == nb - 1)
    def _():
        raw = logits_ref[...]
        gmin = jnp.min(raw)
        avail = mask_ref[...]
        lm = jnp.where(avail, raw, gmin - 1.0)
        logits_ref[...] = lm

        iota_n = jax.lax.broadcasted_iota(jnp.int32, (B, N), 1)
        work = lm
        sel = jnp.zeros((B, N), dtype=jnp.bool_)
        for _ in range(top_k):
            m = jnp.max(work, axis=-1, keepdims=True)
            idx = jnp.min(jnp.where(work == m, iota_n, N), axis=-1, keepdims=True)
            pick = iota_n == idx
            sel = sel | pick
            work = jnp.where(pick, -jnp.inf, work)
        sel = sel & avail

        alpha = alpha_ref[0, 0]
        row_max = jnp.max(lm, axis=-1, keepdims=True)
        e = jnp.where(sel, jnp.exp((lm - row_max) / alpha), 0.0)
        denom = jnp.sum(e, axis=-1, keepdims=True)
        p = e / denom
        log_p = jnp.where(p > 0, jnp.log(jnp.clip(p, 1e-30)), -1e30)
        score = log_p + g_ref[...]
        smax = jnp.max(score, axis=-1, keepdims=True)
        act = jnp.min(jnp.where(score == smax, iota_n, N), axis=-1)
        action_ref[...] = act.reshape(1, B)


def kernel(input_ids, attention_mask, available_mask, a_embeds, embed_table, alpha):
    B, L = input_ids.shape
    _, N, D = a_embeds.shape

    g = jax.random.gumbel(jax.random.key(42), (B, N), jnp.float32)

    w = attention_mask / jnp.maximum(
        attention_mask.sum(axis=-1, keepdims=True), 1e-6)
    w = w.astype(jnp.float32)

    ids_flat = input_ids.reshape(B * L).astype(jnp.int32)
    tok = _sc_gather(embed_table, ids_flat).reshape(B, L, D)

    GB = 4
    logits, action = pl.pallas_call(
        functools.partial(_score_body, B=B, N=N, D=D, GB=GB,
                          top_k=min(N, TOP_K)),
        grid=(B // GB,),
        in_specs=[
            pl.BlockSpec((GB, L, D), lambda b: (b, 0, 0)),
            pl.BlockSpec((GB, 1, L), lambda b: (b, 0, 0)),
            pl.BlockSpec((B, N), lambda b: (0, 0)),
            pl.BlockSpec((B, N), lambda b: (0, 0)),
            pl.BlockSpec(memory_space=pltpu.SMEM),
            pl.BlockSpec((GB, N, D), lambda b: (b, 0, 0)),
        ],
        out_specs=[
            pl.BlockSpec((B, N), lambda b: (0, 0)),
            pl.BlockSpec((1, B), lambda b: (0, 0)),
        ],
        out_shape=[
            jax.ShapeDtypeStruct((B, N), jnp.float32),
            jax.ShapeDtypeStruct((1, B), jnp.int32),
        ],
    )(tok, w.reshape(B, 1, L), available_mask, g, alpha.reshape(1, 1), a_embeds)

    return (action.reshape(B), logits)
